# baseline (speedup 1.0000x reference)
import jax
import jax.numpy as jnp
from jax import lax
from jax.experimental import pallas as pl
from jax.experimental.pallas import tpu as pltpu
from jax.experimental.pallas import tpu_sc as plsc

D = 16
NC, NS = 2, 16
NW = NC * NS
N = 100000
BPW = 3200
LAST = N - (NW - 1) * BPW
CHUNKS_FULL = (512, 512, 512, 512, 512, 512, 128)
CHUNKS_LAST = (512, 288)
CMAX = 512
TROWS = 513


def _run(base, chunks, idx_in_hbm, idx_out_hbm, table_hbm, out_hbm,
         table_v, idx_in_v, idx_out_v, out_v):
    cnt = sum(chunks)
    pltpu.sync_copy(table_hbm, table_v)
    pltpu.sync_copy(idx_in_hbm.at[pl.ds(base, cnt)], idx_in_v.at[pl.ds(0, cnt)])
    pltpu.sync_copy(idx_out_hbm.at[pl.ds(base, cnt)], idx_out_v.at[pl.ds(0, cnt)])

    iota = lax.iota(jnp.int32, 16)

    off = 0
    for ch in chunks:
        o = off

        @plsc.parallel_loop(0, ch // 16, unroll=2)
        def _grp(g):
            j0 = o + g * 16
            iv = idx_in_v[pl.ds(j0, 16)] * D
            ov = idx_out_v[pl.ds(j0, 16)] * D
            for r in range(16):
                rsel = jnp.full((16,), r, jnp.int32)
                a = plsc.load_gather(table_v, [jnp.take(iv, rsel) + iota])
                b = plsc.load_gather(table_v, [jnp.take(ov, rsel) + iota])
                out_v[g * 16 + r, :] = a + b

        pltpu.sync_copy(out_v.at[pl.ds(0, ch)],
                        out_hbm.at[pl.ds(base + off, ch)])
        off += ch


def _body(idx_in_hbm, idx_out_hbm, table_hbm, out_hbm,
          table_v, idx_in_v, idx_out_v, out_v):
    c = lax.axis_index("c")
    s = lax.axis_index("s")
    wid = s * NC + c
    args = (idx_in_hbm, idx_out_hbm, table_hbm, out_hbm,
            table_v, idx_in_v, idx_out_v, out_v)

    @pl.when(wid < NW - 1)
    def _full():
        _run(wid * BPW, CHUNKS_FULL, *args)

    @pl.when(wid == NW - 1)
    def _tail():
        _run((NW - 1) * BPW, CHUNKS_LAST, *args)


@jax.jit
def _degree_encode(idx_in, idx_out, table_flat):
    mesh = plsc.VectorSubcoreMesh(core_axis_name="c", subcore_axis_name="s")
    f = pl.kernel(
        _body,
        out_type=jax.ShapeDtypeStruct((N, D), jnp.float32),
        mesh=mesh,
        scratch_types=[
            pltpu.VMEM((TROWS * D,), jnp.float32),
            pltpu.VMEM((BPW,), jnp.int32),
            pltpu.VMEM((BPW,), jnp.int32),
            pltpu.VMEM((CMAX, D), jnp.float32),
        ],
        compiler_params=pltpu.CompilerParams(needs_layout_passes=False, use_tc_tiling_on_sc=True),
    )
    return f(idx_in, idx_out, table_flat)


def kernel(in_degree, out_degree, table):
    ii = in_degree.astype(jnp.int32)
    oo = out_degree.astype(jnp.int32)
    return _degree_encode(ii, oo, table.reshape(-1))


# speedup vs baseline: 1.0854x; 1.0854x over previous
import jax
import jax.numpy as jnp
from jax import lax
from jax.experimental import pallas as pl
from jax.experimental.pallas import tpu as pltpu
from jax.experimental.pallas import tpu_sc as plsc

D = 16
NC, NS = 2, 16
NW = NC * NS
N = 100000
BPW = 3200
LAST = N - (NW - 1) * BPW
CHUNKS_FULL = (512, 512, 512, 512, 512, 512, 128)
CHUNKS_LAST = (512, 288)
CMAX = 512
TROWS = 513


def _run(base, chunks, idx_in_hbm, idx_out_hbm, table_hbm, out_hbm,
         table_v, idx_in_v, idx_out_v, out_v):
    cnt = sum(chunks)
    pltpu.sync_copy(table_hbm, table_v)
    pltpu.sync_copy(idx_in_hbm.at[pl.ds(base, cnt)], idx_in_v.at[pl.ds(0, cnt)])
    pltpu.sync_copy(idx_out_hbm.at[pl.ds(base, cnt)], idx_out_v.at[pl.ds(0, cnt)])

    iota = lax.iota(jnp.int32, 16)

    off = 0
    for ch in chunks:
        o = off

        @plsc.parallel_loop(0, ch // 16, unroll=4)
        def _grp(g):
            j0 = o + g * 16
            iv = idx_in_v[pl.ds(j0, 16)]
            ov = idx_out_v[pl.ds(j0, 16)]
            for r in range(16):
                rsel = jnp.full((16,), r, jnp.int32)
                a = plsc.load_gather(table_v, [jnp.take(iv, rsel) + iota])
                b = plsc.load_gather(table_v, [jnp.take(ov, rsel) + iota])
                out_v[g * 16 + r, :] = a + b

        pltpu.sync_copy(out_v.at[pl.ds(0, ch)],
                        out_hbm.at[pl.ds(base + off, ch)])
        off += ch


def _body(idx_in_hbm, idx_out_hbm, table_hbm, out_hbm,
          table_v, idx_in_v, idx_out_v, out_v):
    c = lax.axis_index("c")
    s = lax.axis_index("s")
    wid = s * NC + c
    args = (idx_in_hbm, idx_out_hbm, table_hbm, out_hbm,
            table_v, idx_in_v, idx_out_v, out_v)

    @pl.when(wid < NW - 1)
    def _full():
        _run(wid * BPW, CHUNKS_FULL, *args)

    @pl.when(wid == NW - 1)
    def _tail():
        _run((NW - 1) * BPW, CHUNKS_LAST, *args)


@jax.jit
def _degree_encode(idx_in, idx_out, table_flat):
    mesh = plsc.VectorSubcoreMesh(core_axis_name="c", subcore_axis_name="s")
    f = pl.kernel(
        _body,
        out_type=jax.ShapeDtypeStruct((N, D), jnp.float32),
        mesh=mesh,
        scratch_types=[
            pltpu.VMEM((TROWS * D,), jnp.float32),
            pltpu.VMEM((BPW,), jnp.int32),
            pltpu.VMEM((BPW,), jnp.int32),
            pltpu.VMEM((CMAX, D), jnp.float32),
        ],
        compiler_params=pltpu.CompilerParams(needs_layout_passes=False, use_tc_tiling_on_sc=True),
    )
    return f(idx_in, idx_out, table_flat)


def kernel(in_degree, out_degree, table):
    ii = in_degree.astype(jnp.int32) * D
    oo = out_degree.astype(jnp.int32) * D
    return _degree_encode(ii, oo, table.reshape(-1))


# speedup vs baseline: 1.1140x; 1.0264x over previous
import jax
import jax.numpy as jnp
from jax import lax
from jax.experimental import pallas as pl
from jax.experimental.pallas import tpu as pltpu
from jax.experimental.pallas import tpu_sc as plsc

D = 16
NC, NS = 2, 16
NW = NC * NS
N = 100000
BPW = 3200
LAST = N - (NW - 1) * BPW
CHUNKS_FULL = (512, 512, 512, 512, 512, 512, 128)
CHUNKS_LAST = (512, 288)
CMAX = 512
TROWS = 513


def _run(base, chunks, idx_in_hbm, idx_out_hbm, table_hbm, out_hbm,
         table_v, idx_in_v, idx_out_v, out_v):
    cnt = sum(chunks)
    pltpu.sync_copy(table_hbm, table_v)
    pltpu.sync_copy(idx_in_hbm.at[pl.ds(base, cnt)], idx_in_v.at[pl.ds(0, cnt)])
    pltpu.sync_copy(idx_out_hbm.at[pl.ds(base, cnt)], idx_out_v.at[pl.ds(0, cnt)])

    iota = lax.iota(jnp.int32, 16)

    off = 0
    for ch in chunks:
        o = off

        @plsc.parallel_loop(0, ch // 16, unroll=8)
        def _grp(g):
            j0 = o + g * 16
            iv = idx_in_v[pl.ds(j0, 16)]
            ov = idx_out_v[pl.ds(j0, 16)]
            for r in range(16):
                rsel = jnp.full((16,), r, jnp.int32)
                a = plsc.load_gather(table_v, [jnp.take(iv, rsel) + iota])
                b = plsc.load_gather(table_v, [jnp.take(ov, rsel) + iota])
                out_v[g * 16 + r, :] = a + b

        pltpu.sync_copy(out_v.at[pl.ds(0, ch)],
                        out_hbm.at[pl.ds(base + off, ch)])
        off += ch


def _body(idx_in_hbm, idx_out_hbm, table_hbm, out_hbm,
          table_v, idx_in_v, idx_out_v, out_v):
    c = lax.axis_index("c")
    s = lax.axis_index("s")
    wid = s * NC + c
    args = (idx_in_hbm, idx_out_hbm, table_hbm, out_hbm,
            table_v, idx_in_v, idx_out_v, out_v)

    @pl.when(wid < NW - 1)
    def _full():
        _run(wid * BPW, CHUNKS_FULL, *args)

    @pl.when(wid == NW - 1)
    def _tail():
        _run((NW - 1) * BPW, CHUNKS_LAST, *args)


@jax.jit
def _degree_encode(idx_in, idx_out, table_flat):
    mesh = plsc.VectorSubcoreMesh(core_axis_name="c", subcore_axis_name="s")
    f = pl.kernel(
        _body,
        out_type=jax.ShapeDtypeStruct((N, D), jnp.float32),
        mesh=mesh,
        scratch_types=[
            pltpu.VMEM((TROWS * D,), jnp.float32),
            pltpu.VMEM((BPW,), jnp.int32),
            pltpu.VMEM((BPW,), jnp.int32),
            pltpu.VMEM((CMAX, D), jnp.float32),
        ],
        compiler_params=pltpu.CompilerParams(needs_layout_passes=False, use_tc_tiling_on_sc=True),
    )
    return f(idx_in, idx_out, table_flat)


def kernel(in_degree, out_degree, table):
    ii = in_degree.astype(jnp.int32) * D
    oo = out_degree.astype(jnp.int32) * D
    return _degree_encode(ii, oo, table.reshape(-1))
